# baseline (speedup 1.0000x reference)
import jax
import jax.numpy as jnp
from jax import lax
from jax.experimental import pallas as pl
from jax.experimental.pallas import tpu as pltpu
from jax.experimental.pallas import tpu_sc as plsc

B = 16384
K = 128
NC, NS = 2, 16
NW = NC * NS
RPW = B // NW
CH = 128
NCHUNK = RPW // CH
L = 16
NACC = 4


def _body(sid_hbm, qid_hbm, q_hbm, stud_hbm, diff_hbm, disc_hbm, out_hbm,
          sid_v, qid_v, stud_v, diff_v, q_v, disc_v, out_v, sem):
    cid = lax.axis_index("c")
    scid = lax.axis_index("s")
    wid = scid * NC + cid
    lane = lax.broadcasted_iota(jnp.int32, (L,), 0)

    def chunk(c, carry):
        base = wid * RPW + c * CH
        pltpu.sync_copy(sid_hbm.at[pl.ds(base, CH)], sid_v)
        pltpu.sync_copy(qid_hbm.at[pl.ds(base, CH)], qid_v)
        cps = pltpu.async_copy(stud_hbm.at[sid_v], stud_v, sem)
        cpd = pltpu.async_copy(diff_hbm.at[qid_v], diff_v, sem)
        cpc = pltpu.async_copy(disc_hbm.at[qid_v], disc_v, sem)
        pltpu.sync_copy(q_hbm.at[pl.ds(base, CH)], q_v)
        cps.wait()
        cpd.wait()
        cpc.wait()

        def group(g, carry2):
            rows = g * L + lane
            zero = jnp.zeros((L,), jnp.float32)

            def kblock(kk, accs):
                new = list(accs)
                for j in range(L):
                    ks = jnp.full((L,), kk * L + j, jnp.int32)
                    s = plsc.load_gather(stud_v, [rows, ks])
                    d = plsc.load_gather(diff_v, [rows, ks])
                    q = plsc.load_gather(q_v, [rows, ks])
                    es = jnp.exp(s)
                    ed = jnp.exp(d)
                    num = es - ed
                    den = (1.0 + es) * (1.0 + ed)
                    new[j % NACC] = new[j % NACC] + q * (num / den)
                return tuple(new)

            accs = lax.fori_loop(0, K // L, kblock, (zero,) * NACC)
            acc = (accs[0] + accs[1]) + (accs[2] + accs[3])
            dsc = disc_v[pl.ds(g * L, L)]
            sig_dsc = 1.0 / (1.0 + jnp.exp(-dsc))
            x = sig_dsc * acc
            out_v[pl.ds(g * L, L)] = 1.0 / (1.0 + jnp.exp(-x))
            return carry2

        lax.fori_loop(0, CH // L, group, 0)
        pltpu.sync_copy(out_v, out_hbm.at[pl.ds(base, CH)])
        return carry

    lax.fori_loop(0, NCHUNK, chunk, 0)


def kernel(student_id, question, q_matrix_line, student_W, diff_W, disc_W):
    disc_flat = disc_W.reshape(-1)
    mesh = plsc.VectorSubcoreMesh(core_axis_name="c", subcore_axis_name="s")
    f = pl.kernel(
        _body,
        out_type=jax.ShapeDtypeStruct((B,), jnp.float32),
        mesh=mesh,
        compiler_params=pltpu.CompilerParams(needs_layout_passes=False),
        scratch_types=[
            pltpu.VMEM((CH,), jnp.int32),
            pltpu.VMEM((CH,), jnp.int32),
            pltpu.VMEM((CH, K), jnp.float32),
            pltpu.VMEM((CH, K), jnp.float32),
            pltpu.VMEM((CH, K), jnp.float32),
            pltpu.VMEM((CH,), jnp.float32),
            pltpu.VMEM((CH,), jnp.float32),
            pltpu.SemaphoreType.DMA,
        ],
    )
    return f(student_id, question, q_matrix_line, student_W, diff_W, disc_flat)


# speedup vs baseline: 3.0393x; 3.0393x over previous
import jax
import jax.numpy as jnp
from jax import lax
from jax.experimental import pallas as pl
from jax.experimental.pallas import tpu as pltpu
from jax.experimental.pallas import tpu_sc as plsc

B = 16384
K = 128
NC, NS = 2, 16
NW = NC * NS
RPW = B // NW
CH = 128
NCHUNK = RPW // CH
L = 16

_GDN = lax.GatherDimensionNumbers(
    offset_dims=(), collapsed_slice_dims=(0,), start_index_map=(0,))


def _shuffle(x, idx):
    return lax.gather(x, idx[:, None], _GDN, (1,),
                      mode=lax.GatherScatterMode.PROMISE_IN_BOUNDS)


def _body(sid_hbm, qid_hbm, q_hbm, stud_hbm, diff_hbm, disc_hbm, out_hbm,
          sid_v, qid_v, stud_v, diff_v, q_v, disc_v, out_v, sems):
    cid = lax.axis_index("c")
    scid = lax.axis_index("s")
    wid = scid * NC + cid
    lane = lax.broadcasted_iota(jnp.int32, (L,), 0)
    perms = [lane ^ s for s in (8, 4, 2, 1)]

    def issue(c, b):
        base = wid * RPW + c * CH
        pltpu.sync_copy(sid_hbm.at[pl.ds(base, CH)], sid_v.at[b])
        pltpu.sync_copy(qid_hbm.at[pl.ds(base, CH)], qid_v.at[b])
        return (
            pltpu.async_copy(stud_hbm.at[sid_v.at[b]], stud_v.at[b], sems.at[b]),
            pltpu.async_copy(diff_hbm.at[qid_v.at[b]], diff_v.at[b], sems.at[b]),
            pltpu.async_copy(disc_hbm.at[qid_v.at[b]], disc_v.at[b], sems.at[b]),
            pltpu.async_copy(q_hbm.at[pl.ds(base, CH)], q_v.at[b], sems.at[b]),
        )

    def compute(c, b):
        sv, dv, qv = stud_v.at[b], diff_v.at[b], q_v.at[b]

        def group(g, carry):
            def row_fn(r, out_vec):
                row = g * L + r
                a0 = jnp.zeros((L,), jnp.float32)
                a1 = jnp.zeros((L,), jnp.float32)
                for j in range(K // L):
                    s = sv[row, pl.ds(j * L, L)]
                    d = dv[row, pl.ds(j * L, L)]
                    q = qv[row, pl.ds(j * L, L)]
                    es = jnp.exp(s)
                    ed = jnp.exp(d)
                    num = es - ed
                    den = (1.0 + es) * (1.0 + ed)
                    if j % 2 == 0:
                        a0 = a0 + q * (num / den)
                    else:
                        a1 = a1 + q * (num / den)
                acc = a0 + a1
                for p in perms:
                    acc = acc + _shuffle(acc, p)
                return jnp.where(lane == r, acc, out_vec)

            out_vec = lax.fori_loop(0, L, row_fn,
                                    jnp.zeros((L,), jnp.float32))
            dsc = disc_v[b, pl.ds(g * L, L)]
            sig_dsc = 1.0 / (1.0 + jnp.exp(-dsc))
            x = sig_dsc * out_vec
            out_v[pl.ds(g * L, L)] = 1.0 / (1.0 + jnp.exp(-x))
            return carry

        lax.fori_loop(0, CH // L, group, 0)
        base = wid * RPW + c * CH
        pltpu.sync_copy(out_v, out_hbm.at[pl.ds(base, CH)])

    handles = issue(0, 0)
    for c in range(NCHUNK):
        nxt = issue(c + 1, (c + 1) % 2) if c + 1 < NCHUNK else None
        for h in handles:
            h.wait()
        compute(c, c % 2)
        handles = nxt


def kernel(student_id, question, q_matrix_line, student_W, diff_W, disc_W):
    disc_flat = disc_W.reshape(-1)
    mesh = plsc.VectorSubcoreMesh(core_axis_name="c", subcore_axis_name="s")
    f = pl.kernel(
        _body,
        out_type=jax.ShapeDtypeStruct((B,), jnp.float32),
        mesh=mesh,
        compiler_params=pltpu.CompilerParams(needs_layout_passes=False),
        scratch_types=[
            pltpu.VMEM((2, CH), jnp.int32),
            pltpu.VMEM((2, CH), jnp.int32),
            pltpu.VMEM((2, CH, K), jnp.float32),
            pltpu.VMEM((2, CH, K), jnp.float32),
            pltpu.VMEM((2, CH, K), jnp.float32),
            pltpu.VMEM((2, CH), jnp.float32),
            pltpu.VMEM((CH,), jnp.float32),
            pltpu.SemaphoreType.DMA((2,)),
        ],
    )
    return f(student_id, question, q_matrix_line, student_W, diff_W, disc_flat)


# speedup vs baseline: 3.1638x; 1.0410x over previous
import jax
import jax.numpy as jnp
from jax import lax
from jax.experimental import pallas as pl
from jax.experimental.pallas import tpu as pltpu
from jax.experimental.pallas import tpu_sc as plsc

B = 16384
K = 128
NC, NS = 2, 16
NW = NC * NS
RPW = B // NW
CH = 128
NCHUNK = RPW // CH
L = 16

_GDN = lax.GatherDimensionNumbers(
    offset_dims=(), collapsed_slice_dims=(0,), start_index_map=(0,))


def _shuffle(x, idx):
    return lax.gather(x, idx[:, None], _GDN, (1,),
                      mode=lax.GatherScatterMode.PROMISE_IN_BOUNDS)


def _body(sid_hbm, qid_hbm, q_hbm, stud_hbm, diff_hbm, disc_hbm, out_hbm,
          sid_v, qid_v, stud_v, diff_v, q_v, disc_v, out_v, sems):
    cid = lax.axis_index("c")
    scid = lax.axis_index("s")
    wid = scid * NC + cid
    lane = lax.broadcasted_iota(jnp.int32, (L,), 0)
    perms = [lane ^ s for s in (1, 2, 4, 8)]
    masks = [(lane & s) != 0 for s in (1, 2, 4, 8)]

    def issue(c, b):
        base = wid * RPW + c * CH
        pltpu.sync_copy(sid_hbm.at[pl.ds(base, CH)], sid_v.at[b])
        pltpu.sync_copy(qid_hbm.at[pl.ds(base, CH)], qid_v.at[b])
        pltpu.async_copy(stud_hbm.at[sid_v.at[b]], stud_v.at[b], sems.at[b])
        pltpu.async_copy(diff_hbm.at[qid_v.at[b]], diff_v.at[b], sems.at[b])
        pltpu.async_copy(disc_hbm.at[qid_v.at[b]], disc_v.at[b], sems.at[b])
        pltpu.async_copy(q_hbm.at[pl.ds(base, CH)], q_v.at[b], sems.at[b])

    def wait_chunk(b):
        pltpu.make_async_copy(stud_hbm.at[sid_v.at[b]], stud_v.at[b],
                              sems.at[b]).wait()
        pltpu.make_async_copy(diff_hbm.at[qid_v.at[b]], diff_v.at[b],
                              sems.at[b]).wait()
        pltpu.make_async_copy(disc_hbm.at[qid_v.at[b]], disc_v.at[b],
                              sems.at[b]).wait()
        pltpu.make_async_copy(q_hbm.at[pl.ds(0, CH)], q_v.at[b],
                              sems.at[b]).wait()

    def compute(c, b):
        def group(g, carry):
            def half(r0):
                def jstep(j, accs):
                    new = []
                    for r in range(8):
                        row = g * L + r0 + r
                        s = stud_v[b, row, pl.ds(j * L, L)]
                        d = diff_v[b, row, pl.ds(j * L, L)]
                        q = q_v[b, row, pl.ds(j * L, L)]
                        es = jnp.exp(s)
                        ed = jnp.exp(d)
                        num = es - ed
                        den = (1.0 + es) * (1.0 + ed)
                        new.append(accs[r] + q * (num / den))
                    return tuple(new)

                zero = jnp.zeros((L,), jnp.float32)
                accs = lax.fori_loop(0, K // L, jstep, (zero,) * 8)
                level = list(accs)
                for mask, pidx in zip(masks[:3], perms[:3]):
                    nxt = []
                    for i in range(0, len(level), 2):
                        lo, hi = level[i], level[i + 1]
                        nxt.append(jnp.where(mask, _shuffle(hi, pidx), lo)
                                   + jnp.where(mask, hi, _shuffle(lo, pidx)))
                    level = nxt
                return level[0] + _shuffle(level[0], perms[3])

            sums = jnp.where(masks[3], half(8), half(0))
            dsc = disc_v[b, pl.ds(g * L, L)]
            sig_dsc = 1.0 / (1.0 + jnp.exp(-dsc))
            x = sig_dsc * sums
            out_v[pl.ds(g * L, L)] = 1.0 / (1.0 + jnp.exp(-x))
            return carry

        lax.fori_loop(0, CH // L, group, 0)
        base = wid * RPW + c * CH
        pltpu.sync_copy(out_v, out_hbm.at[pl.ds(base, CH)])

    issue(0, 0)

    def chunk_body(c, carry):
        bb = lax.rem(c, 2)

        @pl.when(c + 1 < NCHUNK)
        def _():
            issue(c + 1, lax.rem(c + 1, 2))

        wait_chunk(bb)
        compute(c, bb)
        return carry

    lax.fori_loop(0, NCHUNK, chunk_body, 0)


def kernel(student_id, question, q_matrix_line, student_W, diff_W, disc_W):
    disc_flat = disc_W.reshape(-1)
    mesh = plsc.VectorSubcoreMesh(core_axis_name="c", subcore_axis_name="s")
    f = pl.kernel(
        _body,
        out_type=jax.ShapeDtypeStruct((B,), jnp.float32),
        mesh=mesh,
        compiler_params=pltpu.CompilerParams(needs_layout_passes=False),
        scratch_types=[
            pltpu.VMEM((2, CH), jnp.int32),
            pltpu.VMEM((2, CH), jnp.int32),
            pltpu.VMEM((2, CH, K), jnp.float32),
            pltpu.VMEM((2, CH, K), jnp.float32),
            pltpu.VMEM((2, CH, K), jnp.float32),
            pltpu.VMEM((2, CH), jnp.float32),
            pltpu.VMEM((CH,), jnp.float32),
            pltpu.SemaphoreType.DMA((2,)),
        ],
    )
    return f(student_id, question, q_matrix_line, student_W, diff_W, disc_flat)


# speedup vs baseline: 3.1810x; 1.0054x over previous
import jax
import jax.numpy as jnp
from jax import lax
from jax.experimental import pallas as pl
from jax.experimental.pallas import tpu as pltpu
from jax.experimental.pallas import tpu_sc as plsc

B = 16384
K = 128
NC, NS = 2, 16
NW = NC * NS
RPW = B // NW
CH = 128
NCHUNK = RPW // CH
L = 16

_GDN = lax.GatherDimensionNumbers(
    offset_dims=(), collapsed_slice_dims=(0,), start_index_map=(0,))


def _shuffle(x, idx):
    return lax.gather(x, idx[:, None], _GDN, (1,),
                      mode=lax.GatherScatterMode.PROMISE_IN_BOUNDS)


def _body(sid_hbm, qid_hbm, q_hbm, stud_hbm, diff_hbm, disc_hbm, out_hbm,
          sid_v, qid_v, stud_v, diff_v, q_v, disc_v, out_v, sems):
    cid = lax.axis_index("c")
    scid = lax.axis_index("s")
    wid = scid * NC + cid
    lane = lax.broadcasted_iota(jnp.int32, (L,), 0)
    perms = [lane ^ s for s in (1, 2, 4, 8)]
    masks = [(lane & s) != 0 for s in (1, 2, 4, 8)]

    def issue(c, b):
        base = wid * RPW + c * CH
        pltpu.sync_copy(sid_hbm.at[pl.ds(base, CH)], sid_v.at[b])
        pltpu.sync_copy(qid_hbm.at[pl.ds(base, CH)], qid_v.at[b])
        pltpu.async_copy(stud_hbm.at[sid_v.at[b]], stud_v.at[b], sems.at[b])
        pltpu.async_copy(diff_hbm.at[qid_v.at[b]], diff_v.at[b], sems.at[b])
        pltpu.async_copy(disc_hbm.at[qid_v.at[b]], disc_v.at[b], sems.at[b])
        pltpu.async_copy(q_hbm.at[pl.ds(base, CH)], q_v.at[b], sems.at[b])

    def wait_chunk(b):
        pltpu.make_async_copy(stud_hbm.at[sid_v.at[b]], stud_v.at[b],
                              sems.at[b]).wait()
        pltpu.make_async_copy(diff_hbm.at[qid_v.at[b]], diff_v.at[b],
                              sems.at[b]).wait()
        pltpu.make_async_copy(disc_hbm.at[qid_v.at[b]], disc_v.at[b],
                              sems.at[b]).wait()
        pltpu.make_async_copy(q_hbm.at[pl.ds(0, CH)], q_v.at[b],
                              sems.at[b]).wait()

    def compute(c, b):
        def group(g, carry):
            def jstep(j, accs):
                new = []
                for r in range(L):
                    row = g * L + r
                    s = stud_v[b, row, pl.ds(j * L, L)]
                    d = diff_v[b, row, pl.ds(j * L, L)]
                    q = q_v[b, row, pl.ds(j * L, L)]
                    es = jnp.exp(s)
                    ed = jnp.exp(d)
                    num = es - ed
                    den = (1.0 + es) * (1.0 + ed)
                    new.append(accs[r] + q * (num / den))
                return tuple(new)

            zero = jnp.zeros((L,), jnp.float32)
            level = list(lax.fori_loop(0, K // L, jstep, (zero,) * L))
            for mask, pidx in zip(masks, perms):
                nxt = []
                for i in range(0, len(level), 2):
                    lo, hi = level[i], level[i + 1]
                    nxt.append(jnp.where(mask, _shuffle(hi, pidx), lo)
                               + jnp.where(mask, hi, _shuffle(lo, pidx)))
                level = nxt
            sums = level[0]
            dsc = disc_v[b, pl.ds(g * L, L)]
            sig_dsc = 1.0 / (1.0 + jnp.exp(-dsc))
            x = sig_dsc * sums
            out_v[pl.ds(g * L, L)] = 1.0 / (1.0 + jnp.exp(-x))
            return carry

        lax.fori_loop(0, CH // L, group, 0)
        base = wid * RPW + c * CH
        pltpu.sync_copy(out_v, out_hbm.at[pl.ds(base, CH)])

    issue(0, 0)

    def chunk_body(c, carry):
        bb = lax.rem(c, 2)

        @pl.when(c + 1 < NCHUNK)
        def _():
            issue(c + 1, lax.rem(c + 1, 2))

        wait_chunk(bb)
        compute(c, bb)
        return carry

    lax.fori_loop(0, NCHUNK, chunk_body, 0)


def kernel(student_id, question, q_matrix_line, student_W, diff_W, disc_W):
    disc_flat = disc_W.reshape(-1)
    mesh = plsc.VectorSubcoreMesh(core_axis_name="c", subcore_axis_name="s")
    f = pl.kernel(
        _body,
        out_type=jax.ShapeDtypeStruct((B,), jnp.float32),
        mesh=mesh,
        compiler_params=pltpu.CompilerParams(needs_layout_passes=False),
        scratch_types=[
            pltpu.VMEM((2, CH), jnp.int32),
            pltpu.VMEM((2, CH), jnp.int32),
            pltpu.VMEM((2, CH, K), jnp.float32),
            pltpu.VMEM((2, CH, K), jnp.float32),
            pltpu.VMEM((2, CH, K), jnp.float32),
            pltpu.VMEM((2, CH), jnp.float32),
            pltpu.VMEM((CH,), jnp.float32),
            pltpu.SemaphoreType.DMA((2,)),
        ],
    )
    return f(student_id, question, q_matrix_line, student_W, diff_W, disc_flat)


# speedup vs baseline: 3.3667x; 1.0584x over previous
import jax
import jax.numpy as jnp
from jax import lax
from jax.experimental import pallas as pl
from jax.experimental.pallas import tpu as pltpu
from jax.experimental.pallas import tpu_sc as plsc

B = 16384
K = 128
NC, NS = 2, 16
NW = NC * NS
RPW = B // NW
CH = 128
NCHUNK = RPW // CH
L = 16

_GDN = lax.GatherDimensionNumbers(
    offset_dims=(), collapsed_slice_dims=(0,), start_index_map=(0,))


def _shuffle(x, idx):
    return lax.gather(x, idx[:, None], _GDN, (1,),
                      mode=lax.GatherScatterMode.PROMISE_IN_BOUNDS)


def _body(sid_hbm, qid_hbm, q_hbm, stud_hbm, diff_hbm, disc_hbm, out_hbm,
          sid_all, qid_all, disc_all, stud_v, diff_v, q_v, out_v,
          sems, sem_d):
    cid = lax.axis_index("c")
    scid = lax.axis_index("s")
    wid = scid * NC + cid
    lane = lax.broadcasted_iota(jnp.int32, (L,), 0)
    perms = [lane ^ s for s in (1, 2, 4, 8)]
    masks = [(lane & s) != 0 for s in (1, 2, 4, 8)]

    pltpu.sync_copy(sid_hbm.at[pl.ds(wid * RPW, RPW)], sid_all)
    pltpu.sync_copy(qid_hbm.at[pl.ds(wid * RPW, RPW)], qid_all)
    pltpu.async_copy(disc_hbm.at[qid_all], disc_all, sem_d)

    def issue(c, bb):
        base = wid * RPW + c * CH
        pltpu.async_copy(stud_hbm.at[sid_all.at[pl.ds(c * CH, CH)]],
                         stud_v.at[bb], sems.at[bb])
        pltpu.async_copy(diff_hbm.at[qid_all.at[pl.ds(c * CH, CH)]],
                         diff_v.at[bb], sems.at[bb])
        pltpu.async_copy(q_hbm.at[pl.ds(base, CH)], q_v.at[bb], sems.at[bb])

    def wait_chunk(c, bb):
        pltpu.make_async_copy(stud_hbm.at[sid_all.at[pl.ds(c * CH, CH)]],
                              stud_v.at[bb], sems.at[bb]).wait()
        pltpu.make_async_copy(diff_hbm.at[qid_all.at[pl.ds(c * CH, CH)]],
                              diff_v.at[bb], sems.at[bb]).wait()
        pltpu.make_async_copy(q_hbm.at[pl.ds(0, CH)], q_v.at[bb],
                              sems.at[bb]).wait()

    def compute(c, bb):
        def group(g, carry):
            def jstep(j, accs):
                new = []
                for r in range(L):
                    row = g * L + r
                    s = stud_v[bb, row, pl.ds(j * L, L)]
                    d = diff_v[bb, row, pl.ds(j * L, L)]
                    q = q_v[bb, row, pl.ds(j * L, L)]
                    es = jnp.exp(s)
                    ed = jnp.exp(d)
                    num = es - ed
                    den = (1.0 + es) * (1.0 + ed)
                    new.append(accs[r] + q * (num / den))
                return tuple(new)

            zero = jnp.zeros((L,), jnp.float32)
            level = list(lax.fori_loop(0, K // L, jstep, (zero,) * L))
            for mask, pidx in zip(masks, perms):
                nxt = []
                for i in range(0, len(level), 2):
                    lo, hi = level[i], level[i + 1]
                    nxt.append(jnp.where(mask, _shuffle(hi, pidx), lo)
                               + jnp.where(mask, hi, _shuffle(lo, pidx)))
                level = nxt
            sums = level[0]
            dsc = disc_all[pl.ds(c * CH + g * L, L)]
            sig_dsc = 1.0 / (1.0 + jnp.exp(-dsc))
            x = sig_dsc * sums
            out_v[pl.ds(g * L, L)] = 1.0 / (1.0 + jnp.exp(-x))
            return carry

        lax.fori_loop(0, CH // L, group, 0)
        base = wid * RPW + c * CH
        pltpu.sync_copy(out_v, out_hbm.at[pl.ds(base, CH)])

    issue(0, 0)
    pltpu.make_async_copy(disc_hbm.at[qid_all], disc_all, sem_d).wait()

    def chunk_body(c, carry):
        bb = lax.rem(c, 2)

        @pl.when(c + 1 < NCHUNK)
        def _():
            issue(c + 1, lax.rem(c + 1, 2))

        wait_chunk(c, bb)
        compute(c, bb)
        return carry

    lax.fori_loop(0, NCHUNK, chunk_body, 0)


def kernel(student_id, question, q_matrix_line, student_W, diff_W, disc_W):
    disc_flat = disc_W.reshape(-1)
    mesh = plsc.VectorSubcoreMesh(core_axis_name="c", subcore_axis_name="s")
    f = pl.kernel(
        _body,
        out_type=jax.ShapeDtypeStruct((B,), jnp.float32),
        mesh=mesh,
        compiler_params=pltpu.CompilerParams(needs_layout_passes=False),
        scratch_types=[
            pltpu.VMEM((RPW,), jnp.int32),
            pltpu.VMEM((RPW,), jnp.int32),
            pltpu.VMEM((RPW,), jnp.float32),
            pltpu.VMEM((2, CH, K), jnp.float32),
            pltpu.VMEM((2, CH, K), jnp.float32),
            pltpu.VMEM((2, CH, K), jnp.float32),
            pltpu.VMEM((CH,), jnp.float32),
            pltpu.SemaphoreType.DMA((2,)),
            pltpu.SemaphoreType.DMA,
        ],
    )
    return f(student_id, question, q_matrix_line, student_W, diff_W, disc_flat)
